# baseline (speedup 1.0000x reference)
import functools

import jax
import jax.numpy as jnp
from jax import lax
from jax.experimental import pallas as pl
from jax.experimental.pallas import tpu as pltpu
from jax.experimental.pallas import tpu_sc as plsc

_LANES = 16
_R = 16


@functools.lru_cache(maxsize=None)
def _build(B, S, V, D):
    info = plsc.get_sparse_core_info()
    NC, NS = info.num_cores, info.num_subcores
    NW = NC * NS
    rows_w = B // NW
    n_chunks = rows_w // _R
    assert n_chunks % 2 == 0 and n_chunks >= 4
    goffs = list(range(0, S - _LANES + 1, _LANES))
    if goffs[-1] != S - _LANES:
        goffs.append(S - _LANES)

    mesh = plsc.VectorSubcoreMesh(core_axis_name="c", subcore_axis_name="s")

    @functools.partial(
        pl.kernel,
        out_type=jax.ShapeDtypeStruct((S, D, B), jnp.float32),
        mesh=mesh,
        compiler_params=pltpu.CompilerParams(
            needs_layout_passes=False, use_tc_tiling_on_sc=False),
        scratch_types=[
            pltpu.VMEM((_R, S), jnp.int32),
            pltpu.VMEM((_R, S), jnp.int32),
            pltpu.VMEM((_R, S), jnp.int32),
            pltpu.VMEM((_R, S), jnp.int32),
            pltpu.VMEM((_R, S, D), jnp.float32),
            pltpu.VMEM((_R, S, D), jnp.float32),
            pltpu.VMEM((S, D, _R), jnp.float32),
            pltpu.SemaphoreType.DMA,
            pltpu.SemaphoreType.DMA,
        ],
    )
    def run(idx_hbm, table_hbm, out_hbm, raw0, raw1, safe0, safe1,
            rows0, rows1, rows_t, sem0, sem1):
        wid = lax.axis_index("s") * NC + lax.axis_index("c")
        base = wid * rows_w

        raws = (raw0, raw1)
        safes = (safe0, safe1)
        rows = (rows0, rows1)
        sems = (sem0, sem1)

        z = jnp.zeros((_LANES,), jnp.float32)
        cols = [jnp.full((_LANES,), c, jnp.int32) for c in range(D)]

        def stage_clamp_fire(ci, p):
            raw_v, safe_v, rows_v, sem = raws[p], safes[p], rows[p], sems[p]
            row0 = base + ci * _R
            pltpu.sync_copy(idx_hbm.at[pl.ds(row0, _R)], raw_v)

            def clamp(r, c2):
                for go in goffs:
                    v = raw_v[r, pl.ds(go, _LANES)]
                    safe_v[r, pl.ds(go, _LANES)] = jnp.where(v < V, v, 0)
                return c2
            lax.fori_loop(0, _R, clamp, 0)

            def fire(r, c2):
                pltpu.async_copy(table_hbm.at[safe_v.at[r]], rows_v.at[r], sem)
                return c2
            lax.fori_loop(0, _R, fire, 0)

        def drain(ci, p):
            pltpu.make_async_copy(out_hbm.at[:, :, pl.ds(base + ci * _R, _R)],
                                  rows[p], sems[p]).wait()

        def finish(ci, p):
            raw_v, rows_v = raws[p], rows[p]
            row0 = base + ci * _R

            def fix(r, c2):
                rid = jnp.full((_LANES,), r, jnp.int32)
                for go in goffs:
                    oov = raw_v[r, pl.ds(go, _LANES)] >= V
                    sid = lax.iota(jnp.int32, _LANES) + go
                    for c in range(D):
                        plsc.store_scatter(rows_v, [rid, sid, cols[c]], z,
                                           mask=oov)
                return c2
            lax.fori_loop(0, _R, fix, 0)

            def trans(r, c2):
                rsp = jnp.full((_LANES,), r, jnp.int32)
                for go in goffs:
                    sid = lax.iota(jnp.int32, _LANES) + go
                    for c in range(D):
                        vals = plsc.load_gather(rows_v, [rsp, sid, cols[c]])
                        plsc.store_scatter(rows_t, [sid, cols[c], rsp], vals)
                return c2
            lax.fori_loop(0, _R, trans, 0)

            pltpu.sync_copy(rows_t, out_hbm.at[:, :, pl.ds(row0, _R)])

        def step(ci, p):
            stage_clamp_fire(ci + 1, 1 - p)
            drain(ci, p)
            finish(ci, p)

        stage_clamp_fire(0, 0)

        def pair(g, carry):
            ci = g * 2
            step(ci, 0)
            step(ci + 1, 1)
            return carry
        lax.fori_loop(0, (n_chunks - 2) // 2, pair, 0)

        step(n_chunks - 2, 0)
        drain(n_chunks - 1, 1)
        finish(n_chunks - 1, 1)

    return run


def kernel(indices, table):
    B, S = indices.shape
    V, D = table.shape
    out_t = _build(B, S, V, D)(indices, table)
    return jnp.transpose(out_t, (2, 0, 1))


# speedup vs baseline: 1.3876x; 1.3876x over previous
import functools

import jax
import jax.numpy as jnp
from jax import lax
from jax.experimental import pallas as pl
from jax.experimental.pallas import tpu as pltpu
from jax.experimental.pallas import tpu_sc as plsc

_LANES = 16
_R = 16


@functools.lru_cache(maxsize=None)
def _build(B, S, V, D):
    info = plsc.get_sparse_core_info()
    NC, NS = info.num_cores, info.num_subcores
    NW = NC * NS
    rows_w = B // NW
    n_chunks = rows_w // _R
    assert n_chunks % 2 == 0 and n_chunks >= 4
    goffs = list(range(0, S - _LANES + 1, _LANES))
    if goffs[-1] != S - _LANES:
        goffs.append(S - _LANES)

    mesh = plsc.VectorSubcoreMesh(core_axis_name="c", subcore_axis_name="s")

    @functools.partial(
        pl.kernel,
        out_type=jax.ShapeDtypeStruct((S, B, D), jnp.float32),
        mesh=mesh,
        compiler_params=pltpu.CompilerParams(
            needs_layout_passes=False, use_tc_tiling_on_sc=False),
        scratch_types=[
            pltpu.VMEM((_R, S), jnp.int32),
            pltpu.VMEM((_R, S), jnp.int32),
            pltpu.VMEM((_R, S), jnp.int32),
            pltpu.VMEM((_R, S), jnp.int32),
            pltpu.VMEM((_R, S, D), jnp.float32),
            pltpu.VMEM((_R, S, D), jnp.float32),
            pltpu.VMEM((S, _R, D), jnp.float32),
            pltpu.SemaphoreType.DMA,
            pltpu.SemaphoreType.DMA,
        ],
    )
    def run(idx_hbm, table_hbm, out_hbm, raw0, raw1, safe0, safe1,
            rows0, rows1, rows_t, sem0, sem1):
        wid = lax.axis_index("s") * NC + lax.axis_index("c")
        base = wid * rows_w

        raws = (raw0, raw1)
        safes = (safe0, safe1)
        rows = (rows0, rows1)
        sems = (sem0, sem1)

        z = jnp.zeros((_LANES,), jnp.float32)
        cols = [jnp.full((_LANES,), c, jnp.int32) for c in range(D)]

        def stage_clamp_fire(ci, p):
            raw_v, safe_v, rows_v, sem = raws[p], safes[p], rows[p], sems[p]
            row0 = base + ci * _R
            pltpu.sync_copy(idx_hbm.at[pl.ds(row0, _R)], raw_v)

            def clamp(r, c2):
                for go in goffs:
                    v = raw_v[r, pl.ds(go, _LANES)]
                    safe_v[r, pl.ds(go, _LANES)] = jnp.where(v < V, v, 0)
                return c2
            lax.fori_loop(0, _R, clamp, 0)

            def fire(r, c2):
                pltpu.async_copy(table_hbm.at[safe_v.at[r]], rows_v.at[r], sem)
                return c2
            lax.fori_loop(0, _R, fire, 0)

        def drain(ci, p):
            def one(r, c2):
                pltpu.make_async_copy(out_hbm.at[:, 0, :], rows[p].at[r],
                                      sems[p]).wait()
                return c2
            lax.fori_loop(0, _R, one, 0)

        def finish(ci, p):
            raw_v, rows_v = raws[p], rows[p]
            row0 = base + ci * _R

            def fix(r, c2):
                rid = jnp.full((_LANES,), r, jnp.int32)
                for go in goffs:
                    oov = raw_v[r, pl.ds(go, _LANES)] >= V
                    sid = lax.iota(jnp.int32, _LANES) + go
                    for c in range(D):
                        plsc.store_scatter(rows_v, [rid, sid, cols[c]], z,
                                           mask=oov)
                return c2
            lax.fori_loop(0, _R, fix, 0)

            def trans(r, c2):
                for s in range(S):
                    for h in range(0, D, _LANES):
                        rows_t[s, r, pl.ds(h, _LANES)] = (
                            rows_v[r, s, pl.ds(h, _LANES)])
                return c2
            lax.fori_loop(0, _R, trans, 0)

            pltpu.sync_copy(rows_t, out_hbm.at[:, pl.ds(row0, _R), :])

        def step(ci, p):
            stage_clamp_fire(ci + 1, 1 - p)
            drain(ci, p)
            finish(ci, p)

        stage_clamp_fire(0, 0)

        def pair(g, carry):
            ci = g * 2
            step(ci, 0)
            step(ci + 1, 1)
            return carry
        lax.fori_loop(0, (n_chunks - 2) // 2, pair, 0)

        step(n_chunks - 2, 0)
        drain(n_chunks - 1, 1)
        finish(n_chunks - 1, 1)

    return run


def kernel(indices, table):
    B, S = indices.shape
    V, D = table.shape
    out_t = _build(B, S, V, D)(indices, table)
    return jnp.transpose(out_t, (1, 0, 2))


# speedup vs baseline: 1.3880x; 1.0003x over previous
import functools

import jax
import jax.numpy as jnp
from jax import lax
from jax.experimental import pallas as pl
from jax.experimental.pallas import tpu as pltpu
from jax.experimental.pallas import tpu_sc as plsc

_LANES = 16
_R = 16


@functools.lru_cache(maxsize=None)
def _build(B, S, V, D):
    info = plsc.get_sparse_core_info()
    NC, NS = info.num_cores, info.num_subcores
    NW = NC * NS
    rows_w = B // NW
    n_chunks = rows_w // _R
    assert n_chunks % 2 == 0 and n_chunks >= 4
    goffs = list(range(0, S - _LANES + 1, _LANES))
    if goffs[-1] != S - _LANES:
        goffs.append(S - _LANES)

    mesh = plsc.VectorSubcoreMesh(core_axis_name="c", subcore_axis_name="s")

    @functools.partial(
        pl.kernel,
        out_type=jax.ShapeDtypeStruct((S, B, D), jnp.float32),
        mesh=mesh,
        compiler_params=pltpu.CompilerParams(
            needs_layout_passes=False, use_tc_tiling_on_sc=False),
        scratch_types=[
            pltpu.VMEM((_R, S), jnp.int32),
            pltpu.VMEM((_R, S), jnp.int32),
            pltpu.VMEM((_R * S,), jnp.int32),
            pltpu.VMEM((_R * S,), jnp.int32),
            pltpu.VMEM((_R * S, D), jnp.float32),
            pltpu.VMEM((_R * S, D), jnp.float32),
            pltpu.VMEM((S, _R, D), jnp.float32),
            pltpu.SemaphoreType.DMA,
            pltpu.SemaphoreType.DMA,
        ],
    )
    def run(idx_hbm, table_hbm, out_hbm, raw0, raw1, safe0, safe1,
            rows0, rows1, rows_t, sem0, sem1):
        wid = lax.axis_index("s") * NC + lax.axis_index("c")
        base = wid * rows_w

        raws = (raw0, raw1)
        safes = (safe0, safe1)
        rows = (rows0, rows1)
        sems = (sem0, sem1)

        z = jnp.zeros((_LANES,), jnp.float32)
        cols = [jnp.full((_LANES,), c, jnp.int32) for c in range(D)]

        def stage_clamp_fire(ci, p):
            raw_v, safe_v, rows_v, sem = raws[p], safes[p], rows[p], sems[p]
            row0 = base + ci * _R
            pltpu.sync_copy(idx_hbm.at[pl.ds(row0, _R)], raw_v)

            def clamp(r, c2):
                for go in goffs:
                    v = raw_v[r, pl.ds(go, _LANES)]
                    safe_v[pl.ds(r * S + go, _LANES)] = jnp.where(v < V, v, 0)
                return c2
            lax.fori_loop(0, _R, clamp, 0)

            pltpu.async_copy(table_hbm.at[safe_v], rows_v, sem)

        def drain(ci, p):
            def one(r, c2):
                pltpu.make_async_copy(out_hbm.at[:, 0, :],
                                      rows[p].at[pl.ds(r * S, S)],
                                      sems[p]).wait()
                return c2
            lax.fori_loop(0, _R, one, 0)

        def finish(ci, p):
            raw_v, rows_v = raws[p], rows[p]
            row0 = base + ci * _R

            def fix(r, c2):
                for go in goffs:
                    oov = raw_v[r, pl.ds(go, _LANES)] >= V
                    pid = lax.iota(jnp.int32, _LANES) + (r * S + go)
                    for c in range(D):
                        plsc.store_scatter(rows_v, [pid, cols[c]], z,
                                           mask=oov)
                return c2
            lax.fori_loop(0, _R, fix, 0)

            def trans(r, c2):
                for s in range(S):
                    for h in range(0, D, _LANES):
                        rows_t[s, r, pl.ds(h, _LANES)] = (
                            rows_v[r * S + s, pl.ds(h, _LANES)])
                return c2
            lax.fori_loop(0, _R, trans, 0)

            pltpu.sync_copy(rows_t, out_hbm.at[:, pl.ds(row0, _R), :])

        def step(ci, p):
            stage_clamp_fire(ci + 1, 1 - p)
            drain(ci, p)
            finish(ci, p)

        stage_clamp_fire(0, 0)

        def pair(g, carry):
            ci = g * 2
            step(ci, 0)
            step(ci + 1, 1)
            return carry
        lax.fori_loop(0, (n_chunks - 2) // 2, pair, 0)

        step(n_chunks - 2, 0)
        drain(n_chunks - 1, 1)
        finish(n_chunks - 1, 1)

    return run


def kernel(indices, table):
    B, S = indices.shape
    V, D = table.shape
    out_t = _build(B, S, V, D)(indices, table)
    return jnp.transpose(out_t, (1, 0, 2))
